# baseline (speedup 1.0000x reference)
import functools
import math

import jax
import jax.numpy as jnp
import numpy as np
from jax import lax
from jax.experimental import pallas as pl
from jax.experimental.pallas import tpu as pltpu
from jax.experimental.pallas import tpu_sc as plsc

_VOCAB = 100000
_D = 512
_B = 1024
_S = 50
_SCALE = math.sqrt(_D)
_NW = 32
_N = _B * _S
_W = _B // _NW
_C = 2 * _W
_STEPS = _S // 2
_NBUF = 3
_NV = _D // 16


def _pe_block():
    position = np.arange(0, _S, dtype=np.float32)[:, None]
    div_term = np.exp(
        np.arange(0, _D, 2, dtype=np.float32)
        * np.float32(-(math.log(10000.0) / _D))
    )
    angles = position * div_term
    pe = np.stack([np.sin(angles), np.cos(angles)], axis=-1)
    return jnp.asarray(pe.reshape(_S, _D), dtype=jnp.float32)


@functools.partial(
    pl.kernel,
    mesh=plsc.VectorSubcoreMesh(core_axis_name="c", subcore_axis_name="s"),
    out_type=jax.ShapeDtypeStruct((_N, _D), jnp.float32),
    scratch_types=[
        pltpu.VMEM((_S * _W,), jnp.int32),
        pltpu.VMEM((_C, _D), jnp.float32),
        pltpu.VMEM((_C, _D), jnp.float32),
        pltpu.VMEM((_C, _D), jnp.float32),
        pltpu.VMEM((_S, _D), jnp.float32),
        pltpu.SemaphoreType.DMA,
        pltpu.SemaphoreType.DMA,
        pltpu.SemaphoreType.DMA,
        pltpu.SemaphoreType.DMA,
        pltpu.SemaphoreType.DMA,
        pltpu.SemaphoreType.DMA,
    ],
)
def _emb(table_hbm, idx_hbm, pe_hbm, out_hbm, idx_v, r0, r1, r2,
         pe_v, g0, g1, g2, w0, w1, w2):
    wid = lax.axis_index("s") * 2 + lax.axis_index("c")
    col0 = wid * _W
    pltpu.sync_copy(idx_hbm.at[pl.ds(wid * _S * _W, _S * _W)], idx_v)

    rows = (r0, r1, r2)
    gsem = (g0, g1, g2)
    wsem = (w0, w1, w2)

    def start_gather(g, b):
        off = pl.multiple_of(g * _C, 8)
        pltpu.async_copy(table_hbm.at[idx_v.at[pl.ds(off, _C)]],
                         rows[b], gsem[b])

    def wait_gather(b):
        pltpu.make_async_copy(table_hbm.at[pl.ds(0, _C)], rows[b],
                              gsem[b]).wait()

    def start_write(g, b):
        pltpu.async_copy(
            rows[b].at[pl.ds(0, _W)],
            out_hbm.at[pl.ds(2 * g * _B + col0, _W)], wsem[b])
        pltpu.async_copy(
            rows[b].at[pl.ds(_W, _W)],
            out_hbm.at[pl.ds((2 * g + 1) * _B + col0, _W)], wsem[b])

    def wait_write(b):
        pltpu.make_async_copy(rows[b], out_hbm.at[pl.ds(col0, _C)],
                              wsem[b]).wait()

    def compute(g, b):
        pe_a = [pe_v[2 * g, pl.ds(v * 16, 16)] for v in range(_NV)]
        pe_c = [pe_v[2 * g + 1, pl.ds(v * 16, 16)] for v in range(_NV)]

        def row(i, c2):
            for v in range(_NV):
                sl = pl.ds(v * 16, 16)
                rows[b][i, sl] = rows[b][i, sl] * _SCALE + pe_a[v]
                rows[b][_W + i, sl] = (
                    rows[b][_W + i, sl] * _SCALE + pe_c[v]
                )
            return c2

        lax.fori_loop(0, _W, row, 0)

    start_gather(0, 0)
    start_gather(1, 1)
    pltpu.sync_copy(pe_hbm, pe_v)

    def body(t, carry):
        for k in range(_NBUF):
            g = t * _NBUF + k
            wait_gather(k)

            @pl.when(g + 2 < _STEPS)
            def _issue_next():
                nxt = (k + 2) % _NBUF
                if k >= 1:
                    wait_write(nxt)
                    start_gather(g + 2, nxt)
                else:
                    @pl.when(t >= 1)
                    def _w():
                        wait_write(nxt)

                    start_gather(g + 2, nxt)

            compute(g, k)
            start_write(g, k)
        return carry

    lax.fori_loop(0, _STEPS // _NBUF, body, 0)
    wait_gather(0)
    compute(_STEPS - 1, 0)
    start_write(_STEPS - 1, 0)
    for k in range(_NBUF):
        wait_write(k)


def kernel(x, table):
    idx = (
        x.astype(jnp.int32).T.reshape(_S, _NW, _W)
        .transpose(1, 0, 2).reshape(_N)
    )
    pe = _pe_block()
    out = _emb(table, idx, pe)
    return out.reshape(_S, _B, _D).transpose(1, 0, 2)


# speedup vs baseline: 1.7093x; 1.7093x over previous
import functools
import math

import jax
import jax.numpy as jnp
import numpy as np
from jax import lax
from jax.experimental import pallas as pl
from jax.experimental.pallas import tpu as pltpu
from jax.experimental.pallas import tpu_sc as plsc

_VOCAB = 100000
_D = 512
_B = 1024
_S = 50
_SCALE = math.sqrt(_D)
_NW = 32
_N = _B * _S
_C = _B // _NW
_NBUF = 6
_NV = _D // 16


def _pe_block():
    position = np.arange(0, _S, dtype=np.float32)[:, None]
    div_term = np.exp(
        np.arange(0, _D, 2, dtype=np.float32)
        * np.float32(-(math.log(10000.0) / _D))
    )
    angles = position * div_term
    pe = np.stack([np.sin(angles), np.cos(angles)], axis=-1)
    return jnp.asarray(pe.reshape(_S, _D), dtype=jnp.float32)


@functools.partial(
    pl.kernel,
    mesh=plsc.VectorSubcoreMesh(core_axis_name="c", subcore_axis_name="s"),
    out_type=jax.ShapeDtypeStruct((_N, _D), jnp.float32),
    scratch_types=[
        pltpu.VMEM((_S * _C,), jnp.int32),
        pltpu.VMEM((_C, _D), jnp.float32),
        pltpu.VMEM((_C, _D), jnp.float32),
        pltpu.VMEM((_C, _D), jnp.float32),
        pltpu.VMEM((_C, _D), jnp.float32),
        pltpu.VMEM((_C, _D), jnp.float32),
        pltpu.VMEM((_C, _D), jnp.float32),
        pltpu.VMEM((_S, _D), jnp.float32),
        pltpu.SemaphoreType.DMA,
        pltpu.SemaphoreType.DMA,
        pltpu.SemaphoreType.DMA,
        pltpu.SemaphoreType.DMA,
        pltpu.SemaphoreType.DMA,
        pltpu.SemaphoreType.DMA,
        pltpu.SemaphoreType.DMA,
        pltpu.SemaphoreType.DMA,
        pltpu.SemaphoreType.DMA,
        pltpu.SemaphoreType.DMA,
        pltpu.SemaphoreType.DMA,
        pltpu.SemaphoreType.DMA,
    ],
)
def _emb(table_hbm, idx_hbm, pe_hbm, out_hbm, idx_v, r0, r1, r2, r3, r4,
         r5, pe_v, g0, g1, g2, g3, g4, g5, w0, w1, w2, w3, w4, w5):
    wid = lax.axis_index("s") * 2 + lax.axis_index("c")
    col0 = wid * _C
    pltpu.sync_copy(idx_hbm.at[pl.ds(wid * _S * _C, _S * _C)], idx_v)

    rows = (r0, r1, r2, r3, r4, r5)
    gsem = (g0, g1, g2, g3, g4, g5)
    wsem = (w0, w1, w2, w3, w4, w5)

    def start_gather(g, b):
        off = pl.multiple_of(g * _C, 8)
        pltpu.async_copy(table_hbm.at[idx_v.at[pl.ds(off, _C)]],
                         rows[b], gsem[b])

    def wait_gather(b):
        pltpu.make_async_copy(table_hbm.at[pl.ds(0, _C)], rows[b],
                              gsem[b]).wait()

    def start_write(g, b):
        pltpu.async_copy(rows[b], out_hbm.at[pl.ds(g * _B + col0, _C)],
                         wsem[b])

    def wait_write(b):
        pltpu.make_async_copy(rows[b], out_hbm.at[pl.ds(col0, _C)],
                              wsem[b]).wait()

    def compute(g, b):
        pe_row = [pe_v[g, pl.ds(v * 16, 16)] for v in range(_NV)]

        def row(i, c2):
            for v in range(_NV):
                sl = pl.ds(v * 16, 16)
                rows[b][i, sl] = rows[b][i, sl] * _SCALE + pe_row[v]
            return c2

        lax.fori_loop(0, _C, row, 0)

    start_gather(0, 0)
    start_gather(1, 1)
    start_gather(2, 2)
    pltpu.sync_copy(pe_hbm, pe_v)

    def body(t, carry):
        for k in range(_NBUF):
            g = t * _NBUF + k
            wait_gather(k)

            @pl.when(g + 3 < _S)
            def _issue_next():
                nxt = (k + 3) % _NBUF
                if k >= 3:
                    wait_write(nxt)
                    start_gather(g + 3, nxt)
                else:
                    @pl.when(t >= 1)
                    def _w():
                        wait_write(nxt)

                    start_gather(g + 3, nxt)

            compute(g, k)
            start_write(g, k)
        return carry

    lax.fori_loop(0, _S // _NBUF, body, 0)
    for k in range(_S - (_S // _NBUF) * _NBUF):
        g = (_S // _NBUF) * _NBUF + k
        wait_gather(k)
        compute(g, k)
        start_write(g, k)
    for k in range(_NBUF):
        wait_write(k)


def kernel(x, table):
    idx = (
        x.astype(jnp.int32).T.reshape(_S, _NW, _C)
        .transpose(1, 0, 2).reshape(_N)
    )
    pe = _pe_block()
    out = _emb(table, idx, pe)
    return out.reshape(_S, _B, _D).transpose(1, 0, 2)
